# baseline (speedup 1.0000x reference)
import functools

import jax
import jax.numpy as jnp
from jax import lax
from jax.experimental import pallas as pl
from jax.experimental.pallas import tpu as pltpu
from jax.experimental.pallas import tpu_sc as plsc

NC = 2
NS = 16
NW = NC * NS
L = 16

N_ROWS = 819200
N_TAB = 100000
D = 19
DP = 24

IR = 128
NIR = N_ROWS // IR
IR_PER_W = NIR // NW
IR_PER_T = NIR // NS

UH = 50176
TRASH = UH
ACC_R = 50304
ZROWS = ACC_R // NS
M_ROWS = 2 * UH

A_SUB = 8
A_CHUNKS = IR_PER_T // A_SUB
K_SUB = 8
K_CHUNKS = IR_PER_T // K_SUB
B_CHUNK = 64
B_TOTAL = UH // B_CHUNK
B_PER_T = B_TOTAL // NS
C_SUB = 8
C_CHUNKS = IR_PER_W // C_SUB

_mesh = plsc.VectorSubcoreMesh(core_axis_name="c", subcore_axis_name="s")
_params = pltpu.CompilerParams(use_tc_tiling_on_sc=False,
                               needs_layout_passes=False)


def _worker_id():
    return lax.axis_index("c") * NS + lax.axis_index("s")


def _iota16():
    return lax.iota(jnp.int32, 16)


CNT_R = 100352
CZROWS = CNT_R // NS


@functools.partial(
    pl.kernel,
    out_type=jax.ShapeDtypeStruct((NC, CNT_R), jnp.float32),
    mesh=_mesh,
    compiler_params=_params,
    scratch_types=(
        pltpu.VMEM((K_SUB, IR), jnp.int32),
        pltpu.VMEM((IR,), jnp.float32),
        pltpu.VMEM_SHARED((CNT_R,), jnp.float32),
        pltpu.SemaphoreType.DMA,
    ),
)
def _counts(user_hbm, z1_hbm, cnt_out, ubuf, ones, cnt_sh, sem):
    cid = lax.axis_index("c")
    sid = lax.axis_index("s")
    wid = _worker_id()

    pltpu.sync_copy(z1_hbm, cnt_sh.at[pl.ds(sid * CZROWS, CZROWS)])

    def fill_ones(o, carry):
        ones[pl.ds(o * L, L)] = jnp.full((L,), 1.0, jnp.float32)
        return carry
    lax.fori_loop(0, IR // L, fill_ones, 0)

    plsc.subcore_barrier()

    def chunk_body(c, carry):
        base = wid * IR_PER_W + c * K_SUB
        pltpu.sync_copy(user_hbm.at[pl.ds(base, K_SUB), :], ubuf)
        sdescs = [
            pltpu.async_copy(ones, cnt_sh.at[ubuf.at[j]], sem, add=True)
            for j in range(K_SUB)
        ]
        for dsc in sdescs:
            dsc.wait()
        return carry

    lax.fori_loop(0, IR_PER_W // K_SUB, chunk_body, 0)

    plsc.subcore_barrier()
    pltpu.sync_copy(cnt_sh.at[pl.ds(sid * CZROWS, CZROWS)],
                    cnt_out.at[cid, pl.ds(sid * CZROWS, CZROWS)])


@functools.partial(
    pl.kernel,
    out_type=jax.ShapeDtypeStruct((M_ROWS, D), jnp.float32),
    mesh=_mesh,
    compiler_params=_params,
    scratch_types=(
        pltpu.VMEM((A_SUB, IR), jnp.int32),
        pltpu.VMEM((A_SUB, IR), jnp.int32),
        pltpu.VMEM((A_SUB, IR), jnp.int32),
        pltpu.VMEM((A_SUB, IR, DP), jnp.float32),
        pltpu.VMEM((B_CHUNK, DP), jnp.float32),
        pltpu.VMEM((B_CHUNK,), jnp.float32),
        pltpu.VMEM((B_CHUNK,), jnp.float32),
        pltpu.VMEM((B_CHUNK, D), jnp.float32),
        pltpu.VMEM_SHARED((ACC_R, DP), jnp.float32),
        pltpu.SemaphoreType.DMA,
        pltpu.SemaphoreType.DMA,
    ),
)
def _accumulate(user_hbm, enc_hbm, encp_hbm, cnt_hbm, z2_hbm,
                m_out, ubuf, ebuf, xbuf, rows, abuf, cbuf0, cbuf1, mo,
                acc_sh, sem_g, sem_s):
    cid = lax.axis_index("c")
    sid = lax.axis_index("s")

    pltpu.sync_copy(z2_hbm, acc_sh.at[pl.ds(sid * ZROWS, ZROWS), :])
    plsc.subcore_barrier()

    lo = cid * UH

    def chunk_body(c, carry):
        base = sid * IR_PER_T + c * A_SUB
        pltpu.sync_copy(user_hbm.at[pl.ds(base, A_SUB), :], ubuf)
        pltpu.sync_copy(enc_hbm.at[pl.ds(base, A_SUB), :], ebuf)

        def sel_body(g, carry2):
            j = g // (IR // L)
            off = (g % (IR // L)) * L
            u = ubuf[j, pl.ds(off, L)]
            ul = u - lo
            owned = (ul >= 0) & (ul < UH)
            xbuf[j, pl.ds(off, L)] = jnp.where(
                owned, ul, TRASH + (u & jnp.full((L,), 127, jnp.int32)))
            return carry2
        lax.fori_loop(0, A_SUB * (IR // L), sel_body, 0)

        descs = [
            pltpu.async_copy(encp_hbm.at[ebuf.at[j]], rows.at[j], sem_g)
            for j in range(A_SUB)
        ]
        sdescs = []
        for j, dsc in enumerate(descs):
            dsc.wait()
            sdescs.append(
                pltpu.async_copy(rows.at[j], acc_sh.at[xbuf.at[j]], sem_s,
                                 add=True))
        for dsc in sdescs:
            dsc.wait()
        return carry

    lax.fori_loop(0, A_CHUNKS, chunk_body, 0)

    plsc.subcore_barrier()

    def fin_body(c, carry):
        idx = c * NS + sid

        @pl.when(idx < B_TOTAL)
        def _():
            r0 = idx * B_CHUNK
            pltpu.sync_copy(acc_sh.at[pl.ds(r0, B_CHUNK), :], abuf)
            pltpu.sync_copy(cnt_hbm.at[0, pl.ds(lo + r0, B_CHUNK)], cbuf0)
            pltpu.sync_copy(cnt_hbm.at[1, pl.ds(lo + r0, B_CHUNK)], cbuf1)

            def grp(g, carry2):
                rr = _iota16() + g * L
                r = 1.0 / (1.0 + cbuf0[pl.ds(g * L, L)]
                           + cbuf1[pl.ds(g * L, L)])
                for k in range(D):
                    kk = jnp.full((L,), k, jnp.int32)
                    v = plsc.load_gather(abuf, [rr, kk])
                    plsc.store_scatter(mo, [rr, kk], v * r)
                return carry2
            lax.fori_loop(0, B_CHUNK // L, grp, 0)

            pltpu.sync_copy(mo, m_out.at[pl.ds(lo + r0, B_CHUNK), :])
        return carry

    lax.fori_loop(0, B_PER_T, fin_body, 0)


@functools.partial(
    pl.kernel,
    out_type=jax.ShapeDtypeStruct((NIR, IR), jnp.float32),
    mesh=_mesh,
    compiler_params=_params,
    scratch_types=(
        pltpu.VMEM((C_SUB, IR), jnp.int32),
        pltpu.VMEM((C_SUB, IR), jnp.int32),
        pltpu.VMEM((C_SUB, IR, D), jnp.float32),
        pltpu.VMEM((C_SUB, IR, D), jnp.float32),
        pltpu.VMEM((C_SUB, IR), jnp.float32),
        pltpu.SemaphoreType.DMA,
        pltpu.SemaphoreType.DMA,
    ),
)
def _dot(user_hbm, item_hbm, m_hbm, decp_hbm, out_hbm,
         ubuf, ibuf, urows, irows, ob, sem_u, sem_i):
    wid = _worker_id()

    def chunk_body(c, carry):
        base = wid * IR_PER_W + c * C_SUB
        pltpu.sync_copy(user_hbm.at[pl.ds(base, C_SUB), :], ubuf)
        pltpu.sync_copy(item_hbm.at[pl.ds(base, C_SUB), :], ibuf)

        descs = [
            pltpu.async_copy(m_hbm.at[ubuf.at[j]], urows.at[j], sem_u)
            for j in range(C_SUB)
        ] + [
            pltpu.async_copy(decp_hbm.at[ibuf.at[j]], irows.at[j], sem_i)
            for j in range(C_SUB)
        ]
        for dsc in descs:
            dsc.wait()

        def grp(g, carry2):
            rr = _iota16() + g * L
            jj = rr // IR
            r = rr % IR
            z = jnp.full((L,), 0, jnp.int32)
            acc = (plsc.load_gather(urows, [jj, r, z])
                   + plsc.load_gather(irows, [jj, r, z]))
            for k in range(1, D):
                kk = jnp.full((L,), k, jnp.int32)
                acc = acc + (plsc.load_gather(urows, [jj, r, kk])
                             * plsc.load_gather(irows, [jj, r, kk]))
            plsc.store_scatter(ob, [jj, r], acc)
            return carry2
        lax.fori_loop(0, C_SUB * IR // L, grp, 0)

        pltpu.sync_copy(ob, out_hbm.at[pl.ds(base, C_SUB), :])
        return carry

    lax.fori_loop(0, C_CHUNKS, chunk_body, 0)


def kernel(indices, encoder_bias, encoder_vect, decoder_bias, decoder_vect):
    n = indices.shape[0]
    user2d = indices[:, 0].astype(jnp.int32).reshape(NIR, IR)
    item2d = indices[:, 1].astype(jnp.int32).reshape(NIR, IR)
    enc2d = indices[:, 4].astype(jnp.int32).reshape(NIR, IR)
    mask2d = (jax.random.uniform(jax.random.key(42), (n,)) > 0.5) \
        .astype(jnp.float32).reshape(NIR, IR)
    padt = jnp.zeros((N_TAB, DP - D), jnp.float32)
    encp = jnp.concatenate([encoder_bias, encoder_vect, padt], axis=1)
    decp = jnp.concatenate([decoder_bias, decoder_vect], axis=1)
    z2 = jnp.zeros((ZROWS, DP), jnp.float32)
    z1 = jnp.zeros((CZROWS,), jnp.float32)

    musk2d = jnp.where(mask2d > 0, user2d, user2d + (1 << 20))
    cnt = _counts(user2d, z1)
    mtab = _accumulate(musk2d, enc2d, encp, cnt, z2)
    out = _dot(user2d, item2d, mtab, decp)
    return out.reshape(n, 1)


# speedup vs baseline: 1.0782x; 1.0782x over previous
import functools

import jax
import jax.numpy as jnp
from jax import lax
from jax.experimental import pallas as pl
from jax.experimental.pallas import tpu as pltpu
from jax.experimental.pallas import tpu_sc as plsc

NC = 2
NS = 16
NW = NC * NS
L = 16

N_ROWS = 819200
N_TAB = 100000
D = 19
DP = 24

IR = 128
NIR = N_ROWS // IR
IR_PER_W = NIR // NW
IR_PER_T = NIR // NS

UH = 50176
TRASH = UH
ACC_R = 50304
ZROWS = ACC_R // NS
M_ROWS = 2 * UH

A_SUB = 8
A_CHUNKS = IR_PER_T // A_SUB
K_SUB = 8
K_CHUNKS = IR_PER_T // K_SUB
B_CHUNK = 64
B_TOTAL = UH // B_CHUNK
B_PER_T = B_TOTAL // NS
C_SUB = 8
C_CHUNKS = IR_PER_W // C_SUB

_mesh = plsc.VectorSubcoreMesh(core_axis_name="c", subcore_axis_name="s")
_params = pltpu.CompilerParams(use_tc_tiling_on_sc=False,
                               needs_layout_passes=False)


def _worker_id():
    return lax.axis_index("c") * NS + lax.axis_index("s")


def _iota16():
    return lax.iota(jnp.int32, 16)


CNT_R = 100352
CZROWS = CNT_R // NS


@functools.partial(
    pl.kernel,
    out_type=jax.ShapeDtypeStruct((NC, CNT_R), jnp.float32),
    mesh=_mesh,
    compiler_params=_params,
    scratch_types=(
        pltpu.VMEM((K_SUB, IR), jnp.int32),
        pltpu.VMEM((IR,), jnp.float32),
        pltpu.VMEM_SHARED((CNT_R,), jnp.float32),
        pltpu.SemaphoreType.DMA,
    ),
)
def _counts(user_hbm, z1_hbm, cnt_out, ubuf, ones, cnt_sh, sem):
    cid = lax.axis_index("c")
    sid = lax.axis_index("s")
    wid = _worker_id()

    pltpu.sync_copy(z1_hbm, cnt_sh.at[pl.ds(sid * CZROWS, CZROWS)])

    def fill_ones(o, carry):
        ones[pl.ds(o * L, L)] = jnp.full((L,), 1.0, jnp.float32)
        return carry
    lax.fori_loop(0, IR // L, fill_ones, 0)

    plsc.subcore_barrier()

    def chunk_body(c, carry):
        base = wid * IR_PER_W + c * K_SUB
        pltpu.sync_copy(user_hbm.at[pl.ds(base, K_SUB), :], ubuf)
        sdescs = [
            pltpu.async_copy(ones, cnt_sh.at[ubuf.at[j]], sem, add=True)
            for j in range(K_SUB)
        ]
        for dsc in sdescs:
            dsc.wait()
        return carry

    lax.fori_loop(0, IR_PER_W // K_SUB, chunk_body, 0)

    plsc.subcore_barrier()
    pltpu.sync_copy(cnt_sh.at[pl.ds(sid * CZROWS, CZROWS)],
                    cnt_out.at[cid, pl.ds(sid * CZROWS, CZROWS)])


@functools.partial(
    pl.kernel,
    out_type=jax.ShapeDtypeStruct((M_ROWS, DP), jnp.float32),
    mesh=_mesh,
    compiler_params=_params,
    scratch_types=(
        pltpu.VMEM((A_SUB, IR), jnp.int32),
        pltpu.VMEM((A_SUB, IR), jnp.int32),
        pltpu.VMEM((A_SUB, IR), jnp.int32),
        pltpu.VMEM((A_SUB, IR, DP), jnp.float32),
        pltpu.VMEM((B_CHUNK, DP), jnp.float32),
        pltpu.VMEM((B_CHUNK,), jnp.float32),
        pltpu.VMEM((B_CHUNK,), jnp.float32),
        pltpu.VMEM((B_CHUNK, DP), jnp.float32),
        pltpu.VMEM_SHARED((ACC_R, DP), jnp.float32),
        pltpu.SemaphoreType.DMA,
        pltpu.SemaphoreType.DMA,
    ),
)
def _accumulate(user_hbm, enc_hbm, encp_hbm, cnt_hbm, z2_hbm,
                m_out, ubuf, ebuf, xbuf, rows, abuf, cbuf0, cbuf1, mo,
                acc_sh, sem_g, sem_s):
    cid = lax.axis_index("c")
    sid = lax.axis_index("s")

    pltpu.sync_copy(z2_hbm, acc_sh.at[pl.ds(sid * ZROWS, ZROWS), :])
    plsc.subcore_barrier()

    lo = cid * UH

    def chunk_body(c, carry):
        base = sid * IR_PER_T + c * A_SUB
        pltpu.sync_copy(user_hbm.at[pl.ds(base, A_SUB), :], ubuf)
        pltpu.sync_copy(enc_hbm.at[pl.ds(base, A_SUB), :], ebuf)

        def sel_body(g, carry2):
            j = g // (IR // L)
            off = (g % (IR // L)) * L
            u = ubuf[j, pl.ds(off, L)]
            ul = u - lo
            owned = (ul >= 0) & (ul < UH)
            xbuf[j, pl.ds(off, L)] = jnp.where(
                owned, ul, TRASH + (u & jnp.full((L,), 127, jnp.int32)))
            return carry2
        lax.fori_loop(0, A_SUB * (IR // L), sel_body, 0)

        descs = [
            pltpu.async_copy(encp_hbm.at[ebuf.at[j]], rows.at[j], sem_g)
            for j in range(A_SUB)
        ]
        sdescs = []
        for j, dsc in enumerate(descs):
            dsc.wait()
            sdescs.append(
                pltpu.async_copy(rows.at[j], acc_sh.at[xbuf.at[j]], sem_s,
                                 add=True))
        for dsc in sdescs:
            dsc.wait()
        return carry

    lax.fori_loop(0, A_CHUNKS, chunk_body, 0)

    plsc.subcore_barrier()

    def fin_body(c, carry):
        idx = c * NS + sid

        @pl.when(idx < B_TOTAL)
        def _():
            r0 = idx * B_CHUNK
            pltpu.sync_copy(acc_sh.at[pl.ds(r0, B_CHUNK), :], abuf)
            pltpu.sync_copy(cnt_hbm.at[0, pl.ds(lo + r0, B_CHUNK)], cbuf0)
            pltpu.sync_copy(cnt_hbm.at[1, pl.ds(lo + r0, B_CHUNK)], cbuf1)

            def grp(g, carry2):
                rr = _iota16() + g * L
                r = 1.0 / (1.0 + cbuf0[pl.ds(g * L, L)]
                           + cbuf1[pl.ds(g * L, L)])
                for k in range(D):
                    kk = jnp.full((L,), k, jnp.int32)
                    v = plsc.load_gather(abuf, [rr, kk])
                    plsc.store_scatter(mo, [rr, kk], v * r)
                return carry2
            lax.fori_loop(0, B_CHUNK // L, grp, 0)

            pltpu.sync_copy(mo, m_out.at[pl.ds(lo + r0, B_CHUNK), :])
        return carry

    lax.fori_loop(0, B_PER_T, fin_body, 0)


@functools.partial(
    pl.kernel,
    out_type=jax.ShapeDtypeStruct((NIR, IR), jnp.float32),
    mesh=_mesh,
    compiler_params=_params,
    scratch_types=(
        pltpu.VMEM((2, C_SUB, IR), jnp.int32),
        pltpu.VMEM((2, C_SUB, IR), jnp.int32),
        pltpu.VMEM((2, C_SUB, IR, DP), jnp.float32),
        pltpu.VMEM((2, C_SUB, IR, DP), jnp.float32),
        pltpu.VMEM((2, C_SUB, IR), jnp.float32),
        pltpu.SemaphoreType.DMA,
        pltpu.SemaphoreType.DMA,
        pltpu.SemaphoreType.DMA,
        pltpu.SemaphoreType.DMA,
    ),
)
def _dot(user_hbm, item_hbm, m_hbm, decp_hbm, out_hbm,
         ubuf, ibuf, urows, irows, ob, su0, su1, si0, si1):
    wid = _worker_id()
    sems = ((su0, si0), (su1, si1))

    def issue(c, p):
        base = wid * IR_PER_W + c * C_SUB
        pltpu.sync_copy(user_hbm.at[pl.ds(base, C_SUB), :], ubuf.at[p])
        pltpu.sync_copy(item_hbm.at[pl.ds(base, C_SUB), :], ibuf.at[p])
        for j in range(C_SUB):
            pltpu.async_copy(m_hbm.at[ubuf.at[p, j]], urows.at[p, j],
                             sems[p][0])
            pltpu.async_copy(decp_hbm.at[ibuf.at[p, j]], irows.at[p, j],
                             sems[p][1])

    def drain(p):
        for j in range(C_SUB):
            pltpu.make_async_copy(m_hbm.at[ubuf.at[p, j]], urows.at[p, j],
                                  sems[p][0]).wait()
            pltpu.make_async_copy(decp_hbm.at[ibuf.at[p, j]],
                                  irows.at[p, j], sems[p][1]).wait()

    def compute(c, p):
        def grp(g, carry2):
            rr = _iota16() + g * L
            jj = rr // IR
            r = rr % IR
            z = jnp.full((L,), 0, jnp.int32)
            acc = (plsc.load_gather(urows.at[p], [jj, r, z])
                   + plsc.load_gather(irows.at[p], [jj, r, z]))
            for k in range(1, D):
                kk = jnp.full((L,), k, jnp.int32)
                acc = acc + (plsc.load_gather(urows.at[p], [jj, r, kk])
                             * plsc.load_gather(irows.at[p], [jj, r, kk]))
            plsc.store_scatter(ob.at[p], [jj, r], acc)
            return carry2
        lax.fori_loop(0, C_SUB * IR // L, grp, 0)
        base = wid * IR_PER_W + c * C_SUB
        pltpu.sync_copy(ob.at[p], out_hbm.at[pl.ds(base, C_SUB), :])

    issue(0, 0)
    for c in range(C_CHUNKS):
        p = c & 1
        if c + 1 < C_CHUNKS:
            issue(c + 1, 1 - p)
        drain(p)
        compute(c, p)


def kernel(indices, encoder_bias, encoder_vect, decoder_bias, decoder_vect):
    n = indices.shape[0]
    user2d = indices[:, 0].astype(jnp.int32).reshape(NIR, IR)
    item2d = indices[:, 1].astype(jnp.int32).reshape(NIR, IR)
    enc2d = indices[:, 4].astype(jnp.int32).reshape(NIR, IR)
    mask2d = (jax.random.uniform(jax.random.key(42), (n,)) > 0.5) \
        .astype(jnp.float32).reshape(NIR, IR)
    padt = jnp.zeros((N_TAB, DP - D), jnp.float32)
    encp = jnp.concatenate([encoder_bias, encoder_vect, padt], axis=1)
    decp = jnp.concatenate([decoder_bias, decoder_vect, padt], axis=1)
    z2 = jnp.zeros((ZROWS, DP), jnp.float32)
    z1 = jnp.zeros((CZROWS,), jnp.float32)

    musk2d = jnp.where(mask2d > 0, user2d, user2d + (1 << 20))
    cnt = _counts(user2d, z1)
    mtab = _accumulate(musk2d, enc2d, encp, cnt, z2)
    out = _dot(user2d, item2d, mtab, decp)
    return out.reshape(n, 1)
